# baseline (speedup 1.0000x reference)
import functools

import jax
import jax.numpy as jnp
from jax import lax
from jax.experimental import pallas as pl
from jax.experimental.pallas import tpu as pltpu
from jax.experimental.pallas import tpu_sc as plsc

S = 4
IN = 4096
H = 2048
D = 4
BN = 128
IMG = 768
TXT = 768
B = 1024

BM = 64
NBLK = B // BM + S
P = NBLK * BM
NC, NS = 2, 16
NW = NC * NS
RPW = P // NW

BM_S = 128

f32 = jnp.float32
bf16 = jnp.bfloat16


def _gelu(x):
    return 0.5 * x * (1.0 + jax.lax.erf(x * 0.7071067811865476))


def _ln(x, g, b, eps=1e-5):
    mu = jnp.mean(x, axis=-1, keepdims=True)
    var = jnp.mean((x - mu) ** 2, axis=-1, keepdims=True)
    return (x - mu) * jax.lax.rsqrt(var + eps) * g + b


def _route_body(sid_ref, perm_ref, dst_ref, bsub_ref):
    sid = sid_ref[...]
    i32 = jnp.int32
    lt = (lax.broadcasted_iota(i32, (128, 128), 0)
          <= lax.broadcasted_iota(i32, (128, 128), 1)).astype(bf16)
    q_lt_r = (lax.broadcasted_iota(i32, (8, 8), 1)
              < lax.broadcasted_iota(i32, (8, 8), 0))
    biota = lax.broadcasted_iota(i32, (1, NBLK), 1).astype(f32)
    pos = jnp.zeros((8, 128), f32)
    bsubf = jnp.zeros((1, NBLK), f32)
    ub = jnp.zeros((1, 1), f32)
    for s in range(S):
        m = sid == s
        p = jnp.dot(m.astype(bf16), lt, preferred_element_type=f32)
        t = p[:, 127:128]
        off = jnp.sum(jnp.where(q_lt_r, jnp.reshape(t, (1, 8)), 0.0),
                      axis=1, keepdims=True)
        cnt = off[7:8, :] + t[7:8, :]
        rank = off + p - 1.0
        pos = pos + jnp.where(m, ub * (1.0 * BM) + rank, 0.0)
        bsubf = bsubf + (biota >= ub).astype(f32)
        ub = ub + jnp.floor((cnt + (BM - 1.0)) * (1.0 / BM))
    bsub_ref[...] = (bsubf - 1.0).astype(jnp.int32)
    posr = jnp.reshape(pos, (1, B))
    slots = lax.broadcasted_iota(jnp.int32, (P, 1), 0).astype(f32)
    eq = posr == slots
    src = lax.broadcasted_iota(jnp.int32, (1, B), 1).astype(f32)
    permf = jnp.sum(jnp.where(eq, src, 0.0), axis=1, keepdims=True)
    validf = jnp.sum(eq.astype(f32), axis=1, keepdims=True)
    perm_ref[...] = permf.astype(jnp.int32)
    dst_ref[...] = (permf + (1.0 - validf) * B).astype(jnp.int32)


def _route(sid8):
    return pl.pallas_call(
        _route_body,
        in_specs=[pl.BlockSpec((8, 128), lambda: (0, 0))],
        out_specs=[
            pl.BlockSpec((P, 1), lambda: (0, 0)),
            pl.BlockSpec((P, 1), lambda: (0, 0)),
            pl.BlockSpec((1, NBLK), lambda: (0, 0)),
        ],
        out_shape=[
            jax.ShapeDtypeStruct((P, 1), jnp.int32),
            jax.ShapeDtypeStruct((P, 1), jnp.int32),
            jax.ShapeDtypeStruct((1, NBLK), jnp.int32),
        ],
    )(sid8)


def _gather_body(idx_hbm, vox_hbm, out_hbm, idx_v, rows_v, sem):
    w = lax.axis_index("s") * NC + lax.axis_index("c")
    base = w * RPW
    pltpu.sync_copy(idx_hbm.at[pl.ds(base, RPW)], idx_v)
    pltpu.async_copy(vox_hbm.at[idx_v], rows_v, sem).wait()
    pltpu.sync_copy(rows_v, out_hbm.at[pl.ds(base, RPW), :])


def _sc_gather(perm1d, vox_i32):
    return pl.kernel(
        _gather_body,
        out_type=jax.ShapeDtypeStruct((P, IN // 2), jnp.int32),
        mesh=plsc.VectorSubcoreMesh(core_axis_name="c", subcore_axis_name="s"),
        scratch_types=[
            pltpu.VMEM((RPW,), jnp.int32),
            pltpu.VMEM((RPW, IN // 2), jnp.int32),
            pltpu.SemaphoreType.DMA,
        ],
    )(perm1d, vox_i32)


def _enc_body(sref, x_ref, wd_ref, bd_ref, wu_ref, bu_ref, we_ref, be_ref,
              g_ref, b_ref, out_ref):
    del sref
    x = x_ref[...]
    d = jnp.dot(x, wd_ref[0], preferred_element_type=f32)
    d = _gelu(d + bd_ref[0])
    u = jnp.dot(d.astype(bf16), wu_ref[0], preferred_element_type=f32)
    h = x.astype(f32) + u + bu_ref[0]
    e = jnp.dot(h.astype(bf16), we_ref[0], preferred_element_type=f32)
    e = _ln(e + be_ref[0], g_ref[0], b_ref[0])
    out_ref[...] = _gelu(e).astype(bf16)


def _encoder(bsub, vox_sorted, Wd, bd, Wu, bu, We, be, g_enc, b_enc):
    def xmap(b, sref):
        return (b, 0)

    def wmap(b, sref):
        return (sref[0, b], 0, 0)

    grid_spec = pltpu.PrefetchScalarGridSpec(
        num_scalar_prefetch=1,
        grid=(NBLK,),
        in_specs=[
            pl.BlockSpec((BM, IN), xmap),
            pl.BlockSpec((1, IN, BN), wmap),
            pl.BlockSpec((1, 1, BN), wmap),
            pl.BlockSpec((1, BN, IN), wmap),
            pl.BlockSpec((1, 1, IN), wmap),
            pl.BlockSpec((1, IN, H), wmap),
            pl.BlockSpec((1, 1, H), wmap),
            pl.BlockSpec((1, 1, H), wmap),
            pl.BlockSpec((1, 1, H), wmap),
        ],
        out_specs=pl.BlockSpec((BM, H), xmap),
    )
    return pl.pallas_call(
        _enc_body,
        grid_spec=grid_spec,
        out_shape=jax.ShapeDtypeStruct((P, H), bf16),
        compiler_params=pltpu.CompilerParams(
            dimension_semantics=("arbitrary",)),
    )(bsub, vox_sorted, Wd.astype(bf16), bd.reshape(S, 1, BN),
      Wu.astype(bf16), bu.reshape(S, 1, IN), We.astype(bf16),
      be.reshape(S, 1, H), g_enc.reshape(S, 1, H), b_enc.reshape(S, 1, H))


def _scatter_body(dst_hbm, fs_hbm, out_hbm, idx_v, rows_v, sem):
    w = lax.axis_index("s") * NC + lax.axis_index("c")
    base = w * RPW
    pltpu.sync_copy(dst_hbm.at[pl.ds(base, RPW)], idx_v)
    pltpu.sync_copy(fs_hbm.at[pl.ds(base, RPW), :], rows_v)
    pltpu.async_copy(rows_v, out_hbm.at[idx_v], sem).wait()


def _sc_scatter(dst1d, fs_i32):
    return pl.kernel(
        _scatter_body,
        out_type=jax.ShapeDtypeStruct((B + 1, H // 2), jnp.int32),
        mesh=plsc.VectorSubcoreMesh(core_axis_name="c", subcore_axis_name="s"),
        scratch_types=[
            pltpu.VMEM((RPW,), jnp.int32),
            pltpu.VMEM((RPW, H // 2), jnp.int32),
            pltpu.SemaphoreType.DMA,
        ],
    )(dst1d, fs_i32)


def _stack_body(x_ref, wb_ref, bb_ref, g_ref, b_ref, wi_ref, bi_ref, wt_ref,
                bt_ref, img_ref, txt_ref):
    x = x_ref[...].astype(f32)
    for i in range(D):
        y = jnp.dot(x.astype(bf16), wb_ref[i],
                    preferred_element_type=f32) + bb_ref[i]
        y = _ln(y, g_ref[i], b_ref[i])
        x = x + _gelu(y)
    xb = x.astype(bf16)
    img_ref[...] = jnp.dot(xb, wi_ref[...],
                           preferred_element_type=f32) + bi_ref[...]
    txt_ref[...] = jnp.dot(xb, wt_ref[...],
                           preferred_element_type=f32) + bt_ref[...]


def _stack(feats, Wb, bb, g_bb, b_bb, Wi, bi, Wt, bt):
    nb = B // BM_S
    return pl.pallas_call(
        _stack_body,
        grid=(nb,),
        in_specs=[
            pl.BlockSpec((BM_S, H), lambda b: (b, 0)),
            pl.BlockSpec((D, H, H), lambda b: (0, 0, 0)),
            pl.BlockSpec((D, 1, H), lambda b: (0, 0, 0)),
            pl.BlockSpec((D, 1, H), lambda b: (0, 0, 0)),
            pl.BlockSpec((D, 1, H), lambda b: (0, 0, 0)),
            pl.BlockSpec((H, IMG), lambda b: (0, 0)),
            pl.BlockSpec((1, IMG), lambda b: (0, 0)),
            pl.BlockSpec((H, TXT), lambda b: (0, 0)),
            pl.BlockSpec((1, TXT), lambda b: (0, 0)),
        ],
        out_specs=[
            pl.BlockSpec((BM_S, IMG), lambda b: (b, 0)),
            pl.BlockSpec((BM_S, TXT), lambda b: (b, 0)),
        ],
        out_shape=[
            jax.ShapeDtypeStruct((B, IMG), f32),
            jax.ShapeDtypeStruct((B, TXT), f32),
        ],
        compiler_params=pltpu.CompilerParams(
            dimension_semantics=("arbitrary",)),
    )(feats, Wb.astype(bf16), bb.reshape(D, 1, H), g_bb.reshape(D, 1, H),
      b_bb.reshape(D, 1, H), Wi.astype(bf16), bi.reshape(1, IMG),
      Wt.astype(bf16), bt.reshape(1, TXT))


def kernel(voxels, subject_ids, Wd, bd, Wu, bu, We, be, g_enc, b_enc, Wb, bb,
           g_bb, b_bb, Wi, bi, Wt, bt):
    sid8 = subject_ids.astype(jnp.int32).reshape(8, 128)
    perm, dst, bsub = _route(sid8)
    perm1d = perm.reshape(P)
    dst1d = dst.reshape(P)
    vox_i32 = lax.bitcast_convert_type(
        voxels.astype(bf16).reshape(B, IN // 2, 2), jnp.int32)
    vs_i32 = _sc_gather(perm1d, vox_i32)
    vox_sorted = lax.bitcast_convert_type(vs_i32, bf16).reshape(P, IN)
    feats_sorted = _encoder(bsub, vox_sorted, Wd, bd, Wu, bu, We, be,
                            g_enc, b_enc)
    fs_i32 = lax.bitcast_convert_type(
        feats_sorted.reshape(P, H // 2, 2), jnp.int32)
    feats_i32 = _sc_scatter(dst1d, fs_i32)
    feats = lax.bitcast_convert_type(feats_i32, bf16).reshape(B + 1, H)
    img, txt = _stack(feats, Wb, bb, g_bb, b_bb, Wi, bi, Wt, bt)
    return img, txt


# speedup vs baseline: 1.9585x; 1.9585x over previous
import functools

import jax
import jax.numpy as jnp
from jax import lax
from jax.experimental import pallas as pl
from jax.experimental.pallas import tpu as pltpu
from jax.experimental.pallas import tpu_sc as plsc

S = 4
IN = 4096
H = 2048
D = 4
BN = 128
IMG = 768
TXT = 768
B = 1024

BM = 64
NBLK = B // BM + S
P = NBLK * BM
NC, NS = 2, 16
NW = NC * NS
RPW = P // NW

BM_S = 128

f32 = jnp.float32
bf16 = jnp.bfloat16


def _gelu(x):
    return 0.5 * x * (1.0 + jax.lax.erf(x * 0.7071067811865476))


def _ln(x, g, b, eps=1e-5):
    mu = jnp.mean(x, axis=-1, keepdims=True)
    var = jnp.mean((x - mu) ** 2, axis=-1, keepdims=True)
    return (x - mu) * jax.lax.rsqrt(var + eps) * g + b


def _route_body(sid_ref, perm_ref, dst_ref, bsub_ref):
    sid = sid_ref[...]
    i32 = jnp.int32
    lt = (lax.broadcasted_iota(i32, (128, 128), 0)
          <= lax.broadcasted_iota(i32, (128, 128), 1)).astype(bf16)
    q_lt_r = (lax.broadcasted_iota(i32, (8, 8), 1)
              < lax.broadcasted_iota(i32, (8, 8), 0))
    biota = lax.broadcasted_iota(i32, (1, NBLK), 1).astype(f32)
    pos = jnp.zeros((8, 128), f32)
    bsubf = jnp.zeros((1, NBLK), f32)
    ub = jnp.zeros((1, 1), f32)
    for s in range(S):
        m = sid == s
        p = jnp.dot(m.astype(bf16), lt, preferred_element_type=f32)
        t = p[:, 127:128]
        off = jnp.sum(jnp.where(q_lt_r, jnp.reshape(t, (1, 8)), 0.0),
                      axis=1, keepdims=True)
        cnt = off[7:8, :] + t[7:8, :]
        rank = off + p - 1.0
        pos = pos + jnp.where(m, ub * (1.0 * BM) + rank, 0.0)
        bsubf = bsubf + (biota >= ub).astype(f32)
        ub = ub + jnp.floor((cnt + (BM - 1.0)) * (1.0 / BM))
    bsub_ref[...] = (bsubf - 1.0).astype(jnp.int32)
    posr = jnp.reshape(pos, (1, B))
    slots = lax.broadcasted_iota(jnp.int32, (P, 1), 0).astype(f32)
    eq = posr == slots
    src = lax.broadcasted_iota(jnp.int32, (1, B), 1).astype(f32)
    permf = jnp.sum(jnp.where(eq, src, 0.0), axis=1, keepdims=True)
    validf = jnp.sum(eq.astype(f32), axis=1, keepdims=True)
    perm_ref[...] = permf.astype(jnp.int32)
    dst_ref[...] = (permf + (1.0 - validf) * B).astype(jnp.int32)


def _route(sid8):
    return pl.pallas_call(
        _route_body,
        in_specs=[pl.BlockSpec((8, 128), lambda: (0, 0))],
        out_specs=[
            pl.BlockSpec((P, 1), lambda: (0, 0)),
            pl.BlockSpec((P, 1), lambda: (0, 0)),
            pl.BlockSpec((1, NBLK), lambda: (0, 0)),
        ],
        out_shape=[
            jax.ShapeDtypeStruct((P, 1), jnp.int32),
            jax.ShapeDtypeStruct((P, 1), jnp.int32),
            jax.ShapeDtypeStruct((1, NBLK), jnp.int32),
        ],
    )(sid8)


BPK = 256


def _pack_body(x_ref, out_ref):
    xb = x_ref[...].astype(bf16)
    x2 = jnp.reshape(xb, (2 * BPK, IN // 2))
    out_ref[...] = pltpu.bitcast(x2, jnp.int32)


def _pack(voxels):
    return pl.pallas_call(
        _pack_body,
        grid=(B // BPK,),
        in_specs=[pl.BlockSpec((BPK, IN), lambda b: (b, 0))],
        out_specs=pl.BlockSpec((BPK, IN // 2), lambda b: (b, 0)),
        out_shape=jax.ShapeDtypeStruct((B, IN // 2), jnp.int32),
    )(voxels)


def _gather_body(idx_hbm, vox_hbm, out_hbm, idx_v, rows_v, sem):
    w = lax.axis_index("s") * NC + lax.axis_index("c")
    base = w * RPW
    pltpu.sync_copy(idx_hbm.at[pl.ds(base, RPW)], idx_v)
    pltpu.async_copy(vox_hbm.at[idx_v], rows_v, sem).wait()
    pltpu.sync_copy(rows_v, out_hbm.at[pl.ds(base, RPW), :])


def _sc_gather(perm1d, vox_i32):
    return pl.kernel(
        _gather_body,
        out_type=jax.ShapeDtypeStruct((P, IN // 2), jnp.int32),
        mesh=plsc.VectorSubcoreMesh(core_axis_name="c", subcore_axis_name="s"),
        scratch_types=[
            pltpu.VMEM((RPW,), jnp.int32),
            pltpu.VMEM((RPW, IN // 2), jnp.int32),
            pltpu.SemaphoreType.DMA,
        ],
    )(perm1d, vox_i32)


def _enc_body(sref, x_ref, wd_ref, bd_ref, wu_ref, bu_ref, we_ref, be_ref,
              g_ref, b_ref, out_ref):
    del sref
    x = jnp.reshape(pltpu.bitcast(x_ref[...], bf16), (BM, IN))
    d = jnp.dot(x, wd_ref[0], preferred_element_type=f32)
    d = _gelu(d + bd_ref[0])
    u = jnp.dot(d.astype(bf16), wu_ref[0], preferred_element_type=f32)
    h = x.astype(f32) + u + bu_ref[0]
    e = jnp.dot(h.astype(bf16), we_ref[0], preferred_element_type=f32)
    e = _ln(e + be_ref[0], g_ref[0], b_ref[0])
    eb = jnp.reshape(_gelu(e).astype(bf16), (2 * BM, H // 2))
    out_ref[...] = pltpu.bitcast(eb, jnp.int32)


def _encoder(bsub, vox_sorted, Wd, bd, Wu, bu, We, be, g_enc, b_enc):
    def xmap(b, sref):
        return (b, 0)

    def wmap(b, sref):
        return (sref[0, b], 0, 0)

    grid_spec = pltpu.PrefetchScalarGridSpec(
        num_scalar_prefetch=1,
        grid=(NBLK,),
        in_specs=[
            pl.BlockSpec((BM, IN // 2), xmap),
            pl.BlockSpec((1, IN, BN), wmap),
            pl.BlockSpec((1, 1, BN), wmap),
            pl.BlockSpec((1, BN, IN), wmap),
            pl.BlockSpec((1, 1, IN), wmap),
            pl.BlockSpec((1, IN, H), wmap),
            pl.BlockSpec((1, 1, H), wmap),
            pl.BlockSpec((1, 1, H), wmap),
            pl.BlockSpec((1, 1, H), wmap),
        ],
        out_specs=pl.BlockSpec((BM, H // 2), xmap),
    )
    return pl.pallas_call(
        _enc_body,
        grid_spec=grid_spec,
        out_shape=jax.ShapeDtypeStruct((P, H // 2), jnp.int32),
        compiler_params=pltpu.CompilerParams(
            dimension_semantics=("arbitrary",)),
    )(bsub, vox_sorted, Wd.astype(bf16), bd.reshape(S, 1, BN),
      Wu.astype(bf16), bu.reshape(S, 1, IN), We.astype(bf16),
      be.reshape(S, 1, H), g_enc.reshape(S, 1, H), b_enc.reshape(S, 1, H))


def _scatter_body(dst_hbm, fs_hbm, out_hbm, idx_v, rows_v, sem):
    w = lax.axis_index("s") * NC + lax.axis_index("c")
    base = w * RPW
    pltpu.sync_copy(dst_hbm.at[pl.ds(base, RPW)], idx_v)
    pltpu.sync_copy(fs_hbm.at[pl.ds(base, RPW), :], rows_v)
    pltpu.async_copy(rows_v, out_hbm.at[idx_v], sem).wait()


def _sc_scatter(dst1d, fs_i32):
    return pl.kernel(
        _scatter_body,
        out_type=jax.ShapeDtypeStruct((B + 1, H // 2), jnp.int32),
        mesh=plsc.VectorSubcoreMesh(core_axis_name="c", subcore_axis_name="s"),
        scratch_types=[
            pltpu.VMEM((RPW,), jnp.int32),
            pltpu.VMEM((RPW, H // 2), jnp.int32),
            pltpu.SemaphoreType.DMA,
        ],
    )(dst1d, fs_i32)


def _stack_body(x_ref, wb_ref, bb_ref, g_ref, b_ref, wi_ref, bi_ref, wt_ref,
                bt_ref, img_ref, txt_ref):
    xb = jnp.reshape(pltpu.bitcast(x_ref[...], bf16), (BM_S, H))
    x = xb.astype(f32)
    for i in range(D):
        y = jnp.dot(x.astype(bf16), wb_ref[i],
                    preferred_element_type=f32) + bb_ref[i]
        y = _ln(y, g_ref[i], b_ref[i])
        x = x + _gelu(y)
    xb = x.astype(bf16)
    img_ref[...] = jnp.dot(xb, wi_ref[...],
                           preferred_element_type=f32) + bi_ref[...]
    txt_ref[...] = jnp.dot(xb, wt_ref[...],
                           preferred_element_type=f32) + bt_ref[...]


def _stack(feats, Wb, bb, g_bb, b_bb, Wi, bi, Wt, bt):
    nb = B // BM_S
    return pl.pallas_call(
        _stack_body,
        grid=(nb,),
        in_specs=[
            pl.BlockSpec((BM_S, H // 2), lambda b: (b, 0)),
            pl.BlockSpec((D, H, H), lambda b: (0, 0, 0)),
            pl.BlockSpec((D, 1, H), lambda b: (0, 0, 0)),
            pl.BlockSpec((D, 1, H), lambda b: (0, 0, 0)),
            pl.BlockSpec((D, 1, H), lambda b: (0, 0, 0)),
            pl.BlockSpec((H, IMG), lambda b: (0, 0)),
            pl.BlockSpec((1, IMG), lambda b: (0, 0)),
            pl.BlockSpec((H, TXT), lambda b: (0, 0)),
            pl.BlockSpec((1, TXT), lambda b: (0, 0)),
        ],
        out_specs=[
            pl.BlockSpec((BM_S, IMG), lambda b: (b, 0)),
            pl.BlockSpec((BM_S, TXT), lambda b: (b, 0)),
        ],
        out_shape=[
            jax.ShapeDtypeStruct((B, IMG), f32),
            jax.ShapeDtypeStruct((B, TXT), f32),
        ],
        compiler_params=pltpu.CompilerParams(
            dimension_semantics=("arbitrary",)),
    )(feats, Wb.astype(bf16), bb.reshape(D, 1, H), g_bb.reshape(D, 1, H),
      b_bb.reshape(D, 1, H), Wi.astype(bf16), bi.reshape(1, IMG),
      Wt.astype(bf16), bt.reshape(1, TXT))


def kernel(voxels, subject_ids, Wd, bd, Wu, bu, We, be, g_enc, b_enc, Wb, bb,
           g_bb, b_bb, Wi, bi, Wt, bt):
    sid8 = subject_ids.astype(jnp.int32).reshape(8, 128)
    perm, dst, bsub = _route(sid8)
    perm1d = perm.reshape(P)
    dst1d = dst.reshape(P)
    vox_i32 = _pack(voxels)
    vs_i32 = _sc_gather(perm1d, vox_i32)
    fs_i32 = _encoder(bsub, vs_i32, Wd, bd, Wu, bu, We, be, g_enc, b_enc)
    feats_i32 = _sc_scatter(dst1d, fs_i32)
    img, txt = _stack(feats_i32, Wb, bb, g_bb, b_bb, Wi, bi, Wt, bt)
    return img, txt
